# baseline (speedup 1.0000x reference)
import functools

import jax
import jax.numpy as jnp
from jax import lax
from jax.experimental import pallas as pl
from jax.experimental.pallas import tpu as pltpu
from jax.experimental.pallas import tpu_sc as plsc

_DIM = 2048
_NE = 64
_TILE = 1024
_LANES = 16


def _gate_body(x_ref, w_ref, b_ref, idx_ref, scr_ref):
    x = x_ref[...]
    w = w_ref[...]
    b = b_ref[...]
    logits = lax.dot_general(w, x, (((1,), (1,)), ((), ())),
                             preferred_element_type=jnp.float32) + b
    eid = lax.broadcasted_iota(jnp.int32, logits.shape, 0)
    m1 = jnp.max(logits, axis=0, keepdims=True)
    i1 = jnp.min(jnp.where(logits == m1, eid, _NE), axis=0, keepdims=True)
    masked = jnp.where(eid == i1, -jnp.inf, logits)
    m2 = jnp.max(masked, axis=0, keepdims=True)
    i2 = jnp.min(jnp.where(masked == m2, eid, _NE), axis=0, keepdims=True)
    s1 = 1.0 / (1.0 + jnp.exp(m2 - m1))
    idx_ref[...] = jnp.concatenate([i1, i2], axis=0)
    scr_ref[...] = jnp.concatenate([s1, 1.0 - s1], axis=0)


def _fused_tc(x2, W, b2, n_tok):
    return pl.pallas_call(
        _gate_body,
        grid=(n_tok // _TILE,),
        in_specs=[
            pl.BlockSpec((_TILE, _DIM), lambda i: (i, 0)),
            pl.BlockSpec((_NE, _DIM), lambda i: (0, 0)),
            pl.BlockSpec((_NE, 1), lambda i: (0, 0)),
        ],
        out_specs=[
            pl.BlockSpec((2, _TILE), lambda i: (0, i)),
            pl.BlockSpec((2, _TILE), lambda i: (0, i)),
        ],
        out_shape=[
            jax.ShapeDtypeStruct((2, n_tok), jnp.int32),
            jax.ShapeDtypeStruct((2, n_tok), jnp.float32),
        ],
    )(x2, W, b2)


@functools.partial(
    pl.kernel,
    mesh=plsc.VectorSubcoreMesh(core_axis_name="c", subcore_axis_name="s"),
    out_type=jax.ShapeDtypeStruct((32, _LANES), jnp.float32),
    scratch_types=[pltpu.VMEM((_LANES,), jnp.float32)],
)
def _sc_probe(src_hbm, out_hbm, buf):
    wid = lax.axis_index("s") * 2 + lax.axis_index("c")
    pltpu.sync_copy(src_hbm.at[wid], buf)
    buf[...] = buf[...] * 0.5
    pltpu.sync_copy(buf, out_hbm.at[wid])


def kernel(x, W, b):
    bsz, seq, dim = x.shape
    n_tok = bsz * seq
    x2 = x.reshape(n_tok, dim)
    b2 = b.reshape(_NE, 1)
    dummy = _sc_probe(x2[:32, :_LANES])
    idx_t, scr_t = _fused_tc(x2, W, b2, n_tok)
    z = jnp.min(jnp.abs(dummy)) * 0.0
    idx = idx_t.T.reshape(bsz, seq, 2)
    scr = (scr_t + z).T.reshape(bsz, seq, 2)
    return (idx, scr)
